# Initial kernel scaffold; baseline (speedup 1.0000x reference)
import jax
import jax.numpy as jnp
from jax.experimental import pallas as pl


def kernel(xyz1, xyz2, points1, points2, W1, g1, b1, W2, g2, b2, W3, g3, b3):
    raise NotImplementedError("write your pallas kernel here")


# speedup vs baseline: 9.6795x; 9.6795x over previous
import functools

import jax
import jax.numpy as jnp
from jax import lax
from jax.experimental import pallas as pl
from jax.experimental.pallas import tpu as pltpu
from jax.experimental.pallas import tpu_sc as plsc


def _knn_body(x1_ref, x2t_ref, idx_ref, w_ref, *, S):
    x1 = x1_ref[...]
    x2t = x2t_ref[...]
    n1 = x1[:, 0:1] * x1[:, 0:1] + x1[:, 1:2] * x1[:, 1:2] + x1[:, 2:3] * x1[:, 2:3]
    n2 = x2t[0:1] * x2t[0:1] + x2t[1:2] * x2t[1:2] + x2t[2:3] * x2t[2:3]
    d = -2.0 * jnp.dot(x1, x2t, preferred_element_type=jnp.float32)
    d = d + n1
    d = d + n2
    iota = lax.broadcasted_iota(jnp.int32, d.shape, 1)
    big = jnp.float32(jnp.inf)
    vals, idxs = [], []
    cur = d
    for _ in range(3):
        m = jnp.min(cur, axis=1, keepdims=True)
        im = jnp.min(jnp.where(cur <= m, iota, S), axis=1, keepdims=True)
        vals.append(m)
        idxs.append(im)
        cur = jnp.where(iota == im, big, cur)
    r = [1.0 / (v + 1e-8) for v in vals]
    norm = r[0] + r[1] + r[2]
    b = pl.program_id(0)
    idx_ref[...] = jnp.concatenate(idxs, axis=1).T + b * S
    w_ref[...] = jnp.concatenate([x / norm for x in r], axis=1).T


def _three_nn(xyz1, xyz2, BN=256):
    B, N, _ = xyz1.shape
    S = xyz2.shape[1]
    x2t = jnp.transpose(xyz2, (0, 2, 1))
    NB = N // BN
    idxf, wf = pl.pallas_call(
        functools.partial(_knn_body, S=S),
        grid=(B, NB),
        in_specs=[
            pl.BlockSpec((None, BN, 3), lambda b, i: (b, i, 0)),
            pl.BlockSpec((None, 3, S), lambda b, i: (b, 0, 0)),
        ],
        out_specs=[
            pl.BlockSpec((3, BN), lambda b, i: (0, b * NB + i)),
            pl.BlockSpec((3, BN), lambda b, i: (0, b * NB + i)),
        ],
        out_shape=[
            jax.ShapeDtypeStruct((3, B * N), jnp.int32),
            jax.ShapeDtypeStruct((3, B * N), jnp.float32),
        ],
    )(xyz1, x2t)
    return idxf, wf


def _lane_broadcast(vec, lane_idx):
    return lax.gather(
        vec,
        lane_idx[:, None],
        dimension_numbers=lax.GatherDimensionNumbers(
            offset_dims=(), collapsed_slice_dims=(0,), start_index_map=(0,)
        ),
        slice_sizes=(1,),
        mode=lax.GatherScatterMode.PROMISE_IN_BOUNDS,
    )


def _sc_interpolate(table, idxf, wf):
    BNtot = idxf.shape[1]
    C = table.shape[1]
    NC, NS = 2, 16
    NW = NC * NS
    PW = BNtot // NW
    P = 64
    NCH = PW // P
    CV = C // 16

    mesh = plsc.VectorSubcoreMesh(
        core_axis_name="c", subcore_axis_name="s", num_cores=NC, num_subcores=NS
    )

    @functools.partial(
        pl.kernel,
        mesh=mesh,
        out_type=jax.ShapeDtypeStruct((BNtot, C), jnp.float32),
        scratch_types=[
            pltpu.VMEM((3, PW), jnp.int32),
            pltpu.VMEM((3, PW), jnp.float32),
            pltpu.VMEM((3, P, C), jnp.float32),
            pltpu.VMEM((P, C), jnp.float32),
            pltpu.SemaphoreType.DMA,
        ],
    )
    def interp(table_hbm, idx_hbm, w_hbm, out_hbm, idx_v, w_v, rows_v, out_v, sem):
        wid = lax.axis_index("s") * NC + lax.axis_index("c")
        base = wid * PW
        pltpu.sync_copy(idx_hbm.at[:, pl.ds(base, PW)], idx_v)
        pltpu.sync_copy(w_hbm.at[:, pl.ds(base, PW)], w_v)

        @pl.loop(0, NCH)
        def _chunk(i):
            off = i * P
            cps = [
                pltpu.async_copy(
                    table_hbm.at[idx_v.at[k, pl.ds(off, P)]], rows_v.at[k], sem
                )
                for k in range(3)
            ]
            for cp in cps:
                cp.wait()

            @pl.loop(0, P // 16)
            def _group(g):
                wrow = [w_v[k, pl.ds(off + g * 16, 16)] for k in range(3)]
                for t in range(16):
                    lane = jnp.full((16,), t, jnp.int32)
                    wv = [_lane_broadcast(wrow[k], lane) for k in range(3)]
                    p = g * 16 + t
                    for j in range(CV):
                        sl = pl.ds(j * 16, 16)
                        acc = wv[0] * rows_v[0, p, sl]
                        acc = acc + wv[1] * rows_v[1, p, sl]
                        acc = acc + wv[2] * rows_v[2, p, sl]
                        out_v[p, sl] = acc

            pltpu.sync_copy(out_v, out_hbm.at[pl.ds(base + off, P)])

    return interp(table, idxf, wf)


def _mlp1_body(p1_ref, it_ref, w_ref, z_ref, s_ref):
    x = jnp.concatenate([p1_ref[...], it_ref[...]], axis=1)
    z = jnp.dot(x, w_ref[...], preferred_element_type=jnp.float32)
    z_ref[...] = z

    @pl.when(pl.program_id(0) == 0)
    def _():
        s_ref[...] = jnp.zeros_like(s_ref)

    s_ref[...] += jnp.concatenate(
        [jnp.sum(z, 0, keepdims=True), jnp.sum(z * z, 0, keepdims=True)], axis=0
    )


def _scale_shift(s_ref, g_ref, b_ref, count):
    mean = s_ref[0:1, :] * (1.0 / count)
    ex2 = s_ref[1:2, :] * (1.0 / count)
    var = ex2 - mean * mean
    scale = g_ref[...] * lax.rsqrt(var + 1e-5)
    shift = b_ref[...] - mean * scale
    return scale, shift


def _mlp_mid_body(s_in_ref, g_ref, b_ref, z_in_ref, w_ref, z_ref, s_ref, *, count):
    scale, shift = _scale_shift(s_in_ref, g_ref, b_ref, count)
    a = jnp.maximum(z_in_ref[...] * scale + shift, 0.0)
    z = jnp.dot(a, w_ref[...], preferred_element_type=jnp.float32)
    z_ref[...] = z

    @pl.when(pl.program_id(0) == 0)
    def _():
        s_ref[...] = jnp.zeros_like(s_ref)

    s_ref[...] += jnp.concatenate(
        [jnp.sum(z, 0, keepdims=True), jnp.sum(z * z, 0, keepdims=True)], axis=0
    )


def _final_body(s_in_ref, g_ref, b_ref, z_in_ref, o_ref, *, count):
    scale, shift = _scale_shift(s_in_ref, g_ref, b_ref, count)
    o_ref[...] = jnp.maximum(z_in_ref[...] * scale + shift, 0.0)


def _mlp1(p1, interp, W1t, BM=256):
    BNtot, Ca = p1.shape
    Cb = interp.shape[1]
    Cout = W1t.shape[1]
    NB = BNtot // BM
    return pl.pallas_call(
        _mlp1_body,
        grid=(NB,),
        in_specs=[
            pl.BlockSpec((BM, Ca), lambda i: (i, 0)),
            pl.BlockSpec((BM, Cb), lambda i: (i, 0)),
            pl.BlockSpec((Ca + Cb, Cout), lambda i: (0, 0)),
        ],
        out_specs=[
            pl.BlockSpec((BM, Cout), lambda i: (i, 0)),
            pl.BlockSpec((2, Cout), lambda i: (0, 0)),
        ],
        out_shape=[
            jax.ShapeDtypeStruct((BNtot, Cout), jnp.float32),
            jax.ShapeDtypeStruct((2, Cout), jnp.float32),
        ],
    )(p1, interp, W1t)


def _mlp_mid(s_in, g, b, z_in, Wt, BM=256):
    BNtot, Cin = z_in.shape
    Cout = Wt.shape[1]
    NB = BNtot // BM
    return pl.pallas_call(
        functools.partial(_mlp_mid_body, count=BNtot),
        grid=(NB,),
        in_specs=[
            pl.BlockSpec((2, Cin), lambda i: (0, 0)),
            pl.BlockSpec((1, Cin), lambda i: (0, 0)),
            pl.BlockSpec((1, Cin), lambda i: (0, 0)),
            pl.BlockSpec((BM, Cin), lambda i: (i, 0)),
            pl.BlockSpec((Cin, Cout), lambda i: (0, 0)),
        ],
        out_specs=[
            pl.BlockSpec((BM, Cout), lambda i: (i, 0)),
            pl.BlockSpec((2, Cout), lambda i: (0, 0)),
        ],
        out_shape=[
            jax.ShapeDtypeStruct((BNtot, Cout), jnp.float32),
            jax.ShapeDtypeStruct((2, Cout), jnp.float32),
        ],
    )(s_in, g, b, z_in, Wt)


def _mlp_final(s_in, g, b, z_in, BM=256):
    BNtot, Cin = z_in.shape
    NB = BNtot // BM
    return pl.pallas_call(
        functools.partial(_final_body, count=BNtot),
        grid=(NB,),
        in_specs=[
            pl.BlockSpec((2, Cin), lambda i: (0, 0)),
            pl.BlockSpec((1, Cin), lambda i: (0, 0)),
            pl.BlockSpec((1, Cin), lambda i: (0, 0)),
            pl.BlockSpec((BM, Cin), lambda i: (i, 0)),
        ],
        out_specs=pl.BlockSpec((BM, Cin), lambda i: (i, 0)),
        out_shape=jax.ShapeDtypeStruct((BNtot, Cin), jnp.float32),
    )(s_in, g, b, z_in)


def kernel(xyz1, xyz2, points1, points2, W1, g1, b1, W2, g2, b2, W3, g3, b3):
    B, N, _ = xyz1.shape
    S = xyz2.shape[1]
    C1 = points1.shape[2]
    C2 = points2.shape[2]

    idxf, wf = _three_nn(xyz1, xyz2)

    table = points2.reshape(B * S, C2)
    interp = _sc_interpolate(table, idxf, wf)

    p1 = points1.reshape(B * N, C1)
    z1, s1 = _mlp1(p1, interp, jnp.transpose(W1))
    z2, s2 = _mlp_mid(s1, g1.reshape(1, -1), b1.reshape(1, -1), z1, jnp.transpose(W2))
    z3, s3 = _mlp_mid(s2, g2.reshape(1, -1), b2.reshape(1, -1), z2, jnp.transpose(W3))
    out = _mlp_final(s3, g3.reshape(1, -1), b3.reshape(1, -1), z3)
    return out.reshape(B, N, -1)


# speedup vs baseline: 10.1439x; 1.0480x over previous
import functools

import jax
import jax.numpy as jnp
from jax import lax
from jax.experimental import pallas as pl
from jax.experimental.pallas import tpu as pltpu
from jax.experimental.pallas import tpu_sc as plsc


def _knn_body(x1_ref, x2t_ref, idx_ref, w_ref, *, S):
    x1 = x1_ref[...]
    x2t = x2t_ref[...]
    n1 = x1[:, 0:1] * x1[:, 0:1] + x1[:, 1:2] * x1[:, 1:2] + x1[:, 2:3] * x1[:, 2:3]
    n2 = x2t[0:1] * x2t[0:1] + x2t[1:2] * x2t[1:2] + x2t[2:3] * x2t[2:3]
    d = -2.0 * jnp.dot(x1, x2t, preferred_element_type=jnp.float32)
    d = d + n1
    d = d + n2
    iota = lax.broadcasted_iota(jnp.int32, d.shape, 1)
    big = jnp.float32(jnp.inf)
    vals, idxs = [], []
    cur = d
    for _ in range(3):
        m = jnp.min(cur, axis=1, keepdims=True)
        im = jnp.min(jnp.where(cur <= m, iota, S), axis=1, keepdims=True)
        vals.append(m)
        idxs.append(im)
        cur = jnp.where(iota == im, big, cur)
    r = [1.0 / (v + 1e-8) for v in vals]
    norm = r[0] + r[1] + r[2]
    b = pl.program_id(0)
    idx_ref[...] = jnp.concatenate(idxs, axis=1).T + b * S
    w_ref[...] = jnp.concatenate([x / norm for x in r], axis=1).T


def _three_nn(xyz1, xyz2, BN=256):
    B, N, _ = xyz1.shape
    S = xyz2.shape[1]
    x2t = jnp.transpose(xyz2, (0, 2, 1))
    NB = N // BN
    idxf, wf = pl.pallas_call(
        functools.partial(_knn_body, S=S),
        grid=(B, NB),
        in_specs=[
            pl.BlockSpec((None, BN, 3), lambda b, i: (b, i, 0)),
            pl.BlockSpec((None, 3, S), lambda b, i: (b, 0, 0)),
        ],
        out_specs=[
            pl.BlockSpec((3, BN), lambda b, i: (0, b * NB + i)),
            pl.BlockSpec((3, BN), lambda b, i: (0, b * NB + i)),
        ],
        out_shape=[
            jax.ShapeDtypeStruct((3, B * N), jnp.int32),
            jax.ShapeDtypeStruct((3, B * N), jnp.float32),
        ],
    )(xyz1, x2t)
    return idxf, wf


def _lane_broadcast(vec, lane_idx):
    return lax.gather(
        vec,
        lane_idx[:, None],
        dimension_numbers=lax.GatherDimensionNumbers(
            offset_dims=(), collapsed_slice_dims=(0,), start_index_map=(0,)
        ),
        slice_sizes=(1,),
        mode=lax.GatherScatterMode.PROMISE_IN_BOUNDS,
    )


def _sc_interpolate(table, idxf, wf):
    BNtot = idxf.shape[1]
    C = table.shape[1]
    NC, NS = 2, 16
    NW = NC * NS
    PW = BNtot // NW
    P = 64
    NCH = PW // P
    CV = C // 16

    mesh = plsc.VectorSubcoreMesh(
        core_axis_name="c", subcore_axis_name="s", num_cores=NC, num_subcores=NS
    )

    @functools.partial(
        pl.kernel,
        mesh=mesh,
        out_type=jax.ShapeDtypeStruct((BNtot, C), jnp.float32),
        scratch_types=[
            pltpu.VMEM((3, PW), jnp.int32),
            pltpu.VMEM((3, PW), jnp.float32),
            pltpu.VMEM((3, P, C), jnp.float32),
            pltpu.VMEM((P, C), jnp.float32),
            pltpu.SemaphoreType.DMA,
        ],
    )
    def interp(table_hbm, idx_hbm, w_hbm, out_hbm, idx_v, w_v, rows_v, out_v, sem):
        wid = lax.axis_index("s") * NC + lax.axis_index("c")
        base = wid * PW
        pltpu.sync_copy(idx_hbm.at[:, pl.ds(base, PW)], idx_v)
        pltpu.sync_copy(w_hbm.at[:, pl.ds(base, PW)], w_v)

        @pl.loop(0, NCH)
        def _chunk(i):
            off = i * P
            cps = [
                pltpu.async_copy(
                    table_hbm.at[idx_v.at[k, pl.ds(off, P)]], rows_v.at[k], sem
                )
                for k in range(3)
            ]
            for cp in cps:
                cp.wait()

            @pl.loop(0, P // 16)
            def _group(g):
                wrow = [w_v[k, pl.ds(off + g * 16, 16)] for k in range(3)]
                for t in range(16):
                    lane = jnp.full((16,), t, jnp.int32)
                    wv = [_lane_broadcast(wrow[k], lane) for k in range(3)]
                    p = g * 16 + t
                    for j in range(CV):
                        sl = pl.ds(j * 16, 16)
                        acc = wv[0] * rows_v[0, p, sl]
                        acc = acc + wv[1] * rows_v[1, p, sl]
                        acc = acc + wv[2] * rows_v[2, p, sl]
                        out_v[p, sl] = acc

            pltpu.sync_copy(out_v, out_hbm.at[pl.ds(base + off, P)])

    return interp(table, idxf, wf)


def _mlp1_body(p1_ref, it_ref, w_ref, z_ref, s_ref):
    x = jnp.concatenate([p1_ref[...], it_ref[...]], axis=1)
    z = jnp.dot(x, w_ref[...], preferred_element_type=jnp.float32)
    z_ref[...] = z.astype(z_ref.dtype)

    @pl.when(pl.program_id(0) == 0)
    def _():
        s_ref[...] = jnp.zeros_like(s_ref)

    s_ref[...] += jnp.concatenate(
        [jnp.sum(z, 0, keepdims=True), jnp.sum(z * z, 0, keepdims=True)], axis=0
    )


def _scale_shift(s_ref, g_ref, b_ref, count):
    mean = s_ref[0:1, :] * (1.0 / count)
    ex2 = s_ref[1:2, :] * (1.0 / count)
    var = ex2 - mean * mean
    scale = g_ref[...] * lax.rsqrt(var + 1e-5)
    shift = b_ref[...] - mean * scale
    return scale, shift


def _mlp_mid_body(s_in_ref, g_ref, b_ref, z_in_ref, w_ref, z_ref, s_ref, *, count):
    scale, shift = _scale_shift(s_in_ref, g_ref, b_ref, count)
    a = jnp.maximum(z_in_ref[...].astype(jnp.float32) * scale + shift, 0.0)
    z = jnp.dot(a, w_ref[...], preferred_element_type=jnp.float32)
    z_ref[...] = z.astype(z_ref.dtype)

    @pl.when(pl.program_id(0) == 0)
    def _():
        s_ref[...] = jnp.zeros_like(s_ref)

    s_ref[...] += jnp.concatenate(
        [jnp.sum(z, 0, keepdims=True), jnp.sum(z * z, 0, keepdims=True)], axis=0
    )


def _final_body(s_in_ref, g_ref, b_ref, z_in_ref, o_ref, *, count):
    scale, shift = _scale_shift(s_in_ref, g_ref, b_ref, count)
    o_ref[...] = jnp.maximum(z_in_ref[...].astype(jnp.float32) * scale + shift, 0.0)


def _mlp1(p1, interp, W1t, BM=256):
    BNtot, Ca = p1.shape
    Cb = interp.shape[1]
    Cout = W1t.shape[1]
    NB = BNtot // BM
    return pl.pallas_call(
        _mlp1_body,
        grid=(NB,),
        in_specs=[
            pl.BlockSpec((BM, Ca), lambda i: (i, 0)),
            pl.BlockSpec((BM, Cb), lambda i: (i, 0)),
            pl.BlockSpec((Ca + Cb, Cout), lambda i: (0, 0)),
        ],
        out_specs=[
            pl.BlockSpec((BM, Cout), lambda i: (i, 0)),
            pl.BlockSpec((2, Cout), lambda i: (0, 0)),
        ],
        out_shape=[
            jax.ShapeDtypeStruct((BNtot, Cout), jnp.bfloat16),
            jax.ShapeDtypeStruct((2, Cout), jnp.float32),
        ],
    )(p1, interp, W1t)


def _mlp_mid(s_in, g, b, z_in, Wt, BM=256):
    BNtot, Cin = z_in.shape
    Cout = Wt.shape[1]
    NB = BNtot // BM
    return pl.pallas_call(
        functools.partial(_mlp_mid_body, count=BNtot),
        grid=(NB,),
        in_specs=[
            pl.BlockSpec((2, Cin), lambda i: (0, 0)),
            pl.BlockSpec((1, Cin), lambda i: (0, 0)),
            pl.BlockSpec((1, Cin), lambda i: (0, 0)),
            pl.BlockSpec((BM, Cin), lambda i: (i, 0)),
            pl.BlockSpec((Cin, Cout), lambda i: (0, 0)),
        ],
        out_specs=[
            pl.BlockSpec((BM, Cout), lambda i: (i, 0)),
            pl.BlockSpec((2, Cout), lambda i: (0, 0)),
        ],
        out_shape=[
            jax.ShapeDtypeStruct((BNtot, Cout), jnp.bfloat16),
            jax.ShapeDtypeStruct((2, Cout), jnp.float32),
        ],
    )(s_in, g, b, z_in, Wt)


def _mlp_final(s_in, g, b, z_in, BM=256):
    BNtot, Cin = z_in.shape
    NB = BNtot // BM
    return pl.pallas_call(
        functools.partial(_final_body, count=BNtot),
        grid=(NB,),
        in_specs=[
            pl.BlockSpec((2, Cin), lambda i: (0, 0)),
            pl.BlockSpec((1, Cin), lambda i: (0, 0)),
            pl.BlockSpec((1, Cin), lambda i: (0, 0)),
            pl.BlockSpec((BM, Cin), lambda i: (i, 0)),
        ],
        out_specs=pl.BlockSpec((BM, Cin), lambda i: (i, 0)),
        out_shape=jax.ShapeDtypeStruct((BNtot, Cin), jnp.float32),
    )(s_in, g, b, z_in)


def kernel(xyz1, xyz2, points1, points2, W1, g1, b1, W2, g2, b2, W3, g3, b3):
    B, N, _ = xyz1.shape
    S = xyz2.shape[1]
    C1 = points1.shape[2]
    C2 = points2.shape[2]

    idxf, wf = _three_nn(xyz1, xyz2)

    table = points2.reshape(B * S, C2)
    interp = _sc_interpolate(table, idxf, wf)

    p1 = points1.reshape(B * N, C1)
    z1, s1 = _mlp1(p1, interp, jnp.transpose(W1))
    z2, s2 = _mlp_mid(s1, g1.reshape(1, -1), b1.reshape(1, -1), z1, jnp.transpose(W2))
    z3, s3 = _mlp_mid(s2, g2.reshape(1, -1), b2.reshape(1, -1), z2, jnp.transpose(W3))
    out = _mlp_final(s3, g3.reshape(1, -1), b3.reshape(1, -1), z3)
    return out.reshape(B, N, -1)


# speedup vs baseline: 13.1575x; 1.2971x over previous
import functools

import jax
import jax.numpy as jnp
from jax import lax
from jax.experimental import pallas as pl
from jax.experimental.pallas import tpu as pltpu
from jax.experimental.pallas import tpu_sc as plsc


def _knn_body(x1_ref, x2t_ref, idx_ref, w_ref, *, S):
    x1 = x1_ref[...]
    x2t = x2t_ref[...]
    n1 = x1[:, 0:1] * x1[:, 0:1] + x1[:, 1:2] * x1[:, 1:2] + x1[:, 2:3] * x1[:, 2:3]
    n2 = x2t[0:1] * x2t[0:1] + x2t[1:2] * x2t[1:2] + x2t[2:3] * x2t[2:3]
    d = -2.0 * jnp.dot(x1, x2t, preferred_element_type=jnp.float32)
    d = d + n1
    d = d + n2
    iota = lax.broadcasted_iota(jnp.int32, d.shape, 1)
    big = jnp.float32(jnp.inf)
    vals, idxs = [], []
    cur = d
    for _ in range(3):
        m = jnp.min(cur, axis=1, keepdims=True)
        im = jnp.min(jnp.where(cur <= m, iota, S), axis=1, keepdims=True)
        vals.append(m)
        idxs.append(im)
        cur = jnp.where(iota == im, big, cur)
    r = [1.0 / (v + 1e-8) for v in vals]
    norm = r[0] + r[1] + r[2]
    b = pl.program_id(0)
    idx_ref[...] = jnp.concatenate(idxs, axis=1).T + b * S
    w_ref[...] = jnp.concatenate([x / norm for x in r], axis=1).T


def _three_nn(xyz1, xyz2, BN=512):
    B, N, _ = xyz1.shape
    S = xyz2.shape[1]
    x2t = jnp.transpose(xyz2, (0, 2, 1))
    NB = N // BN
    idxf, wf = pl.pallas_call(
        functools.partial(_knn_body, S=S),
        grid=(B, NB),
        in_specs=[
            pl.BlockSpec((None, BN, 3), lambda b, i: (b, i, 0)),
            pl.BlockSpec((None, 3, S), lambda b, i: (b, 0, 0)),
        ],
        out_specs=[
            pl.BlockSpec((3, BN), lambda b, i: (0, b * NB + i)),
            pl.BlockSpec((3, BN), lambda b, i: (0, b * NB + i)),
        ],
        out_shape=[
            jax.ShapeDtypeStruct((3, B * N), jnp.int32),
            jax.ShapeDtypeStruct((3, B * N), jnp.float32),
        ],
    )(xyz1, x2t)
    return idxf, wf


def _lane_broadcast(vec, lane_idx):
    return lax.gather(
        vec,
        lane_idx[:, None],
        dimension_numbers=lax.GatherDimensionNumbers(
            offset_dims=(), collapsed_slice_dims=(0,), start_index_map=(0,)
        ),
        slice_sizes=(1,),
        mode=lax.GatherScatterMode.PROMISE_IN_BOUNDS,
    )


def _sc_interpolate(table, idxf, wf):
    BNtot = idxf.shape[1]
    C = table.shape[1]
    NC, NS = 2, 16
    NW = NC * NS
    PW = BNtot // NW
    P = 64
    NCH = PW // P
    CV = C // 16

    mesh = plsc.VectorSubcoreMesh(
        core_axis_name="c", subcore_axis_name="s", num_cores=NC, num_subcores=NS
    )

    @functools.partial(
        pl.kernel,
        mesh=mesh,
        out_type=jax.ShapeDtypeStruct((BNtot, C), jnp.float32),
        scratch_types=[
            pltpu.VMEM((3, PW), jnp.int32),
            pltpu.VMEM((3, PW), jnp.float32),
            pltpu.VMEM((3, P, C), jnp.float32),
            pltpu.VMEM((P, C), jnp.float32),
            pltpu.SemaphoreType.DMA,
        ],
    )
    def interp(table_hbm, idx_hbm, w_hbm, out_hbm, idx_v, w_v, rows_v, out_v, sem):
        wid = lax.axis_index("s") * NC + lax.axis_index("c")
        base = wid * PW
        pltpu.sync_copy(idx_hbm.at[:, pl.ds(base, PW)], idx_v)
        pltpu.sync_copy(w_hbm.at[:, pl.ds(base, PW)], w_v)

        @pl.loop(0, NCH)
        def _chunk(i):
            off = i * P
            cps = [
                pltpu.async_copy(
                    table_hbm.at[idx_v.at[k, pl.ds(off, P)]], rows_v.at[k], sem
                )
                for k in range(3)
            ]
            for cp in cps:
                cp.wait()

            @pl.loop(0, P // 16)
            def _group(g):
                wrow = [w_v[k, pl.ds(off + g * 16, 16)] for k in range(3)]
                for t in range(16):
                    lane = jnp.full((16,), t, jnp.int32)
                    wv = [_lane_broadcast(wrow[k], lane) for k in range(3)]
                    p = g * 16 + t
                    for j in range(CV):
                        sl = pl.ds(j * 16, 16)
                        acc = wv[0] * rows_v[0, p, sl]
                        acc = acc + wv[1] * rows_v[1, p, sl]
                        acc = acc + wv[2] * rows_v[2, p, sl]
                        out_v[p, sl] = acc

            pltpu.sync_copy(out_v, out_hbm.at[pl.ds(base + off, P)])

    return interp(table, idxf, wf)


def _mlp1_body(p1_ref, it_ref, w_ref, z_ref, s_ref):
    x = jnp.concatenate([p1_ref[...], it_ref[...]], axis=1)
    z = jnp.dot(x, w_ref[...], preferred_element_type=jnp.float32)
    z_ref[...] = z.astype(z_ref.dtype)

    @pl.when(pl.program_id(0) == 0)
    def _():
        s_ref[...] = jnp.zeros_like(s_ref)

    s_ref[...] += jnp.concatenate(
        [jnp.sum(z, 0, keepdims=True), jnp.sum(z * z, 0, keepdims=True)], axis=0
    )


def _scale_shift(s_ref, g_ref, b_ref, count):
    mean = s_ref[0:1, :] * (1.0 / count)
    ex2 = s_ref[1:2, :] * (1.0 / count)
    var = ex2 - mean * mean
    scale = g_ref[...] * lax.rsqrt(var + 1e-5)
    shift = b_ref[...] - mean * scale
    return scale, shift


def _mlp_mid_body(s_in_ref, g_ref, b_ref, z_in_ref, w_ref, z_ref, s_ref, *, count):
    scale, shift = _scale_shift(s_in_ref, g_ref, b_ref, count)
    a = jnp.maximum(z_in_ref[...].astype(jnp.float32) * scale + shift, 0.0)
    z = jnp.dot(a, w_ref[...], preferred_element_type=jnp.float32)
    z_ref[...] = z.astype(z_ref.dtype)

    @pl.when(pl.program_id(0) == 0)
    def _():
        s_ref[...] = jnp.zeros_like(s_ref)

    s_ref[...] += jnp.concatenate(
        [jnp.sum(z, 0, keepdims=True), jnp.sum(z * z, 0, keepdims=True)], axis=0
    )


def _final_body(s_in_ref, g_ref, b_ref, z_in_ref, o_ref, *, count):
    scale, shift = _scale_shift(s_in_ref, g_ref, b_ref, count)
    o_ref[...] = jnp.maximum(z_in_ref[...].astype(jnp.float32) * scale + shift, 0.0)


def _mlp1(p1, interp, W1t, BM=512):
    BNtot, Ca = p1.shape
    Cb = interp.shape[1]
    Cout = W1t.shape[1]
    NB = BNtot // BM
    return pl.pallas_call(
        _mlp1_body,
        grid=(NB,),
        in_specs=[
            pl.BlockSpec((BM, Ca), lambda i: (i, 0)),
            pl.BlockSpec((BM, Cb), lambda i: (i, 0)),
            pl.BlockSpec((Ca + Cb, Cout), lambda i: (0, 0)),
        ],
        out_specs=[
            pl.BlockSpec((BM, Cout), lambda i: (i, 0)),
            pl.BlockSpec((2, Cout), lambda i: (0, 0)),
        ],
        out_shape=[
            jax.ShapeDtypeStruct((BNtot, Cout), jnp.bfloat16),
            jax.ShapeDtypeStruct((2, Cout), jnp.float32),
        ],
    )(p1, interp, W1t)


def _mlp_mid(s_in, g, b, z_in, Wt, BM=512):
    BNtot, Cin = z_in.shape
    Cout = Wt.shape[1]
    NB = BNtot // BM
    return pl.pallas_call(
        functools.partial(_mlp_mid_body, count=BNtot),
        grid=(NB,),
        in_specs=[
            pl.BlockSpec((2, Cin), lambda i: (0, 0)),
            pl.BlockSpec((1, Cin), lambda i: (0, 0)),
            pl.BlockSpec((1, Cin), lambda i: (0, 0)),
            pl.BlockSpec((BM, Cin), lambda i: (i, 0)),
            pl.BlockSpec((Cin, Cout), lambda i: (0, 0)),
        ],
        out_specs=[
            pl.BlockSpec((BM, Cout), lambda i: (i, 0)),
            pl.BlockSpec((2, Cout), lambda i: (0, 0)),
        ],
        out_shape=[
            jax.ShapeDtypeStruct((BNtot, Cout), jnp.bfloat16),
            jax.ShapeDtypeStruct((2, Cout), jnp.float32),
        ],
    )(s_in, g, b, z_in, Wt)


def _mlp_final(s_in, g, b, z_in, BM=512):
    BNtot, Cin = z_in.shape
    NB = BNtot // BM
    return pl.pallas_call(
        functools.partial(_final_body, count=BNtot),
        grid=(NB,),
        in_specs=[
            pl.BlockSpec((2, Cin), lambda i: (0, 0)),
            pl.BlockSpec((1, Cin), lambda i: (0, 0)),
            pl.BlockSpec((1, Cin), lambda i: (0, 0)),
            pl.BlockSpec((BM, Cin), lambda i: (i, 0)),
        ],
        out_specs=pl.BlockSpec((BM, Cin), lambda i: (i, 0)),
        out_shape=jax.ShapeDtypeStruct((BNtot, Cin), jnp.float32),
    )(s_in, g, b, z_in)


def kernel(xyz1, xyz2, points1, points2, W1, g1, b1, W2, g2, b2, W3, g3, b3):
    B, N, _ = xyz1.shape
    S = xyz2.shape[1]
    C1 = points1.shape[2]
    C2 = points2.shape[2]

    idxf, wf = _three_nn(xyz1, xyz2)

    table = points2.reshape(B * S, C2)
    interp = _sc_interpolate(table, idxf, wf)

    p1 = points1.reshape(B * N, C1)
    z1, s1 = _mlp1(p1, interp, jnp.transpose(W1))
    z2, s2 = _mlp_mid(s1, g1.reshape(1, -1), b1.reshape(1, -1), z1, jnp.transpose(W2))
    z3, s3 = _mlp_mid(s2, g2.reshape(1, -1), b2.reshape(1, -1), z2, jnp.transpose(W3))
    out = _mlp_final(s3, g3.reshape(1, -1), b3.reshape(1, -1), z3)
    return out.reshape(B, N, -1)


# speedup vs baseline: 15.1835x; 1.1540x over previous
import functools

import jax
import jax.numpy as jnp
from jax import lax
from jax.experimental import pallas as pl
from jax.experimental.pallas import tpu as pltpu
from jax.experimental.pallas import tpu_sc as plsc


def _knn_body(x1_ref, x2t_ref, idx_ref, w_ref, *, S):
    x1 = x1_ref[...]
    x2t = x2t_ref[...]
    n1 = x1[:, 0:1] * x1[:, 0:1] + x1[:, 1:2] * x1[:, 1:2] + x1[:, 2:3] * x1[:, 2:3]
    n2 = x2t[0:1] * x2t[0:1] + x2t[1:2] * x2t[1:2] + x2t[2:3] * x2t[2:3]
    d = -2.0 * jnp.dot(x1, x2t, preferred_element_type=jnp.float32)
    d = d + n1
    d = d + n2
    iota = lax.broadcasted_iota(jnp.int32, d.shape, 1)
    big = jnp.float32(jnp.inf)
    vals, idxs = [], []
    cur = d
    for _ in range(3):
        m = jnp.min(cur, axis=1, keepdims=True)
        im = jnp.min(jnp.where(cur <= m, iota, S), axis=1, keepdims=True)
        vals.append(m)
        idxs.append(im)
        cur = jnp.where(iota == im, big, cur)
    r = [1.0 / (v + 1e-8) for v in vals]
    norm = r[0] + r[1] + r[2]
    b = pl.program_id(0)
    idx_ref[...] = jnp.concatenate(idxs, axis=1).T + b * S
    w_ref[...] = jnp.concatenate([x / norm for x in r], axis=1).T


def _three_nn(xyz1, xyz2, BN=1024):
    B, N, _ = xyz1.shape
    S = xyz2.shape[1]
    x2t = jnp.transpose(xyz2, (0, 2, 1))
    NB = N // BN
    idxf, wf = pl.pallas_call(
        functools.partial(_knn_body, S=S),
        grid=(B, NB),
        in_specs=[
            pl.BlockSpec((None, BN, 3), lambda b, i: (b, i, 0)),
            pl.BlockSpec((None, 3, S), lambda b, i: (b, 0, 0)),
        ],
        out_specs=[
            pl.BlockSpec((3, BN), lambda b, i: (0, b * NB + i)),
            pl.BlockSpec((3, BN), lambda b, i: (0, b * NB + i)),
        ],
        out_shape=[
            jax.ShapeDtypeStruct((3, B * N), jnp.int32),
            jax.ShapeDtypeStruct((3, B * N), jnp.float32),
        ],
    )(xyz1, x2t)
    return idxf, wf


def _lane_broadcast(vec, lane_idx):
    return lax.gather(
        vec,
        lane_idx[:, None],
        dimension_numbers=lax.GatherDimensionNumbers(
            offset_dims=(), collapsed_slice_dims=(0,), start_index_map=(0,)
        ),
        slice_sizes=(1,),
        mode=lax.GatherScatterMode.PROMISE_IN_BOUNDS,
    )


def _sc_interpolate(table, idxf, wf):
    BNtot = idxf.shape[1]
    C = table.shape[1]
    NC, NS = 2, 16
    NW = NC * NS
    PW = BNtot // NW
    P = 64
    NCH = PW // P
    CV = C // 16

    mesh = plsc.VectorSubcoreMesh(
        core_axis_name="c", subcore_axis_name="s", num_cores=NC, num_subcores=NS
    )

    @functools.partial(
        pl.kernel,
        mesh=mesh,
        out_type=jax.ShapeDtypeStruct((BNtot, C), jnp.float32),
        scratch_types=[
            pltpu.VMEM((3, PW), jnp.int32),
            pltpu.VMEM((3, PW), jnp.float32),
            pltpu.VMEM((3, P, C), jnp.float32),
            pltpu.VMEM((P, C), jnp.float32),
            pltpu.SemaphoreType.DMA,
        ],
    )
    def interp(table_hbm, idx_hbm, w_hbm, out_hbm, idx_v, w_v, rows_v, out_v, sem):
        wid = lax.axis_index("s") * NC + lax.axis_index("c")
        base = wid * PW
        pltpu.sync_copy(idx_hbm.at[:, pl.ds(base, PW)], idx_v)
        pltpu.sync_copy(w_hbm.at[:, pl.ds(base, PW)], w_v)

        @pl.loop(0, NCH)
        def _chunk(i):
            off = i * P
            cps = [
                pltpu.async_copy(
                    table_hbm.at[idx_v.at[k, pl.ds(off, P)]], rows_v.at[k], sem
                )
                for k in range(3)
            ]
            for cp in cps:
                cp.wait()

            @pl.loop(0, P // 16)
            def _group(g):
                wrow = [w_v[k, pl.ds(off + g * 16, 16)] for k in range(3)]
                for t in range(16):
                    lane = jnp.full((16,), t, jnp.int32)
                    wv = [_lane_broadcast(wrow[k], lane) for k in range(3)]
                    p = g * 16 + t
                    for j in range(CV):
                        sl = pl.ds(j * 16, 16)
                        acc = wv[0] * rows_v[0, p, sl]
                        acc = acc + wv[1] * rows_v[1, p, sl]
                        acc = acc + wv[2] * rows_v[2, p, sl]
                        out_v[p, sl] = acc

            pltpu.sync_copy(out_v, out_hbm.at[pl.ds(base + off, P)])

    return interp(table, idxf, wf)


def _mlp1_body(p1_ref, it_ref, w_ref, z_ref, s_ref):
    x = jnp.concatenate([p1_ref[...], it_ref[...]], axis=1)
    z = jnp.dot(x, w_ref[...], preferred_element_type=jnp.float32)
    z_ref[...] = z.astype(z_ref.dtype)

    @pl.when(pl.program_id(0) == 0)
    def _():
        s_ref[...] = jnp.zeros_like(s_ref)

    s_ref[...] += jnp.concatenate(
        [jnp.sum(z, 0, keepdims=True), jnp.sum(z * z, 0, keepdims=True)], axis=0
    )


def _scale_shift(s_ref, g_ref, b_ref, count):
    mean = s_ref[0:1, :] * (1.0 / count)
    ex2 = s_ref[1:2, :] * (1.0 / count)
    var = ex2 - mean * mean
    scale = g_ref[...] * lax.rsqrt(var + 1e-5)
    shift = b_ref[...] - mean * scale
    return scale, shift


def _mlp_mid_body(s_in_ref, g_ref, b_ref, z_in_ref, w_ref, z_ref, s_ref, *, count):
    scale, shift = _scale_shift(s_in_ref, g_ref, b_ref, count)
    a = jnp.maximum(z_in_ref[...].astype(jnp.float32) * scale + shift, 0.0)
    z = jnp.dot(a, w_ref[...], preferred_element_type=jnp.float32)
    z_ref[...] = z.astype(z_ref.dtype)

    @pl.when(pl.program_id(0) == 0)
    def _():
        s_ref[...] = jnp.zeros_like(s_ref)

    s_ref[...] += jnp.concatenate(
        [jnp.sum(z, 0, keepdims=True), jnp.sum(z * z, 0, keepdims=True)], axis=0
    )


def _final_body(s_in_ref, g_ref, b_ref, z_in_ref, o_ref, *, count):
    scale, shift = _scale_shift(s_in_ref, g_ref, b_ref, count)
    o_ref[...] = jnp.maximum(z_in_ref[...].astype(jnp.float32) * scale + shift, 0.0)


def _mlp1(p1, interp, W1t, BM=1024):
    BNtot, Ca = p1.shape
    Cb = interp.shape[1]
    Cout = W1t.shape[1]
    NB = BNtot // BM
    return pl.pallas_call(
        _mlp1_body,
        grid=(NB,),
        in_specs=[
            pl.BlockSpec((BM, Ca), lambda i: (i, 0)),
            pl.BlockSpec((BM, Cb), lambda i: (i, 0)),
            pl.BlockSpec((Ca + Cb, Cout), lambda i: (0, 0)),
        ],
        out_specs=[
            pl.BlockSpec((BM, Cout), lambda i: (i, 0)),
            pl.BlockSpec((2, Cout), lambda i: (0, 0)),
        ],
        out_shape=[
            jax.ShapeDtypeStruct((BNtot, Cout), jnp.bfloat16),
            jax.ShapeDtypeStruct((2, Cout), jnp.float32),
        ],
    )(p1, interp, W1t)


def _mlp_mid(s_in, g, b, z_in, Wt, BM=1024):
    BNtot, Cin = z_in.shape
    Cout = Wt.shape[1]
    NB = BNtot // BM
    return pl.pallas_call(
        functools.partial(_mlp_mid_body, count=BNtot),
        grid=(NB,),
        in_specs=[
            pl.BlockSpec((2, Cin), lambda i: (0, 0)),
            pl.BlockSpec((1, Cin), lambda i: (0, 0)),
            pl.BlockSpec((1, Cin), lambda i: (0, 0)),
            pl.BlockSpec((BM, Cin), lambda i: (i, 0)),
            pl.BlockSpec((Cin, Cout), lambda i: (0, 0)),
        ],
        out_specs=[
            pl.BlockSpec((BM, Cout), lambda i: (i, 0)),
            pl.BlockSpec((2, Cout), lambda i: (0, 0)),
        ],
        out_shape=[
            jax.ShapeDtypeStruct((BNtot, Cout), jnp.bfloat16),
            jax.ShapeDtypeStruct((2, Cout), jnp.float32),
        ],
    )(s_in, g, b, z_in, Wt)


def _mlp_final(s_in, g, b, z_in, BM=1024):
    BNtot, Cin = z_in.shape
    NB = BNtot // BM
    return pl.pallas_call(
        functools.partial(_final_body, count=BNtot),
        grid=(NB,),
        in_specs=[
            pl.BlockSpec((2, Cin), lambda i: (0, 0)),
            pl.BlockSpec((1, Cin), lambda i: (0, 0)),
            pl.BlockSpec((1, Cin), lambda i: (0, 0)),
            pl.BlockSpec((BM, Cin), lambda i: (i, 0)),
        ],
        out_specs=pl.BlockSpec((BM, Cin), lambda i: (i, 0)),
        out_shape=jax.ShapeDtypeStruct((BNtot, Cin), jnp.float32),
    )(s_in, g, b, z_in)


def kernel(xyz1, xyz2, points1, points2, W1, g1, b1, W2, g2, b2, W3, g3, b3):
    B, N, _ = xyz1.shape
    S = xyz2.shape[1]
    C1 = points1.shape[2]
    C2 = points2.shape[2]

    idxf, wf = _three_nn(xyz1, xyz2)

    table = points2.reshape(B * S, C2)
    interp = _sc_interpolate(table, idxf, wf)

    p1 = points1.reshape(B * N, C1)
    z1, s1 = _mlp1(p1, interp, jnp.transpose(W1))
    z2, s2 = _mlp_mid(s1, g1.reshape(1, -1), b1.reshape(1, -1), z1, jnp.transpose(W2))
    z3, s3 = _mlp_mid(s2, g2.reshape(1, -1), b2.reshape(1, -1), z2, jnp.transpose(W3))
    out = _mlp_final(s3, g3.reshape(1, -1), b3.reshape(1, -1), z3)
    return out.reshape(B, N, -1)


# speedup vs baseline: 16.5295x; 1.0886x over previous
import functools

import jax
import jax.numpy as jnp
from jax import lax
from jax.experimental import pallas as pl
from jax.experimental.pallas import tpu as pltpu
from jax.experimental.pallas import tpu_sc as plsc


def _knn_body(x1_ref, x2t_ref, idx_ref, w_ref, *, S):
    x1 = x1_ref[...]
    x2t = x2t_ref[...]
    n1 = x1[:, 0:1] * x1[:, 0:1] + x1[:, 1:2] * x1[:, 1:2] + x1[:, 2:3] * x1[:, 2:3]
    n2 = x2t[0:1] * x2t[0:1] + x2t[1:2] * x2t[1:2] + x2t[2:3] * x2t[2:3]
    d = -2.0 * jnp.dot(x1, x2t, preferred_element_type=jnp.float32)
    d = d + n1
    d = d + n2
    iota = lax.broadcasted_iota(jnp.int32, d.shape, 1)
    big = jnp.float32(jnp.inf)
    vals, idxs = [], []
    cur = d
    for _ in range(3):
        m = jnp.min(cur, axis=1, keepdims=True)
        im = jnp.min(jnp.where(cur <= m, iota, S), axis=1, keepdims=True)
        vals.append(m)
        idxs.append(im)
        cur = jnp.where(iota == im, big, cur)
    r = [1.0 / (v + 1e-8) for v in vals]
    norm = r[0] + r[1] + r[2]
    b = pl.program_id(0)
    idx_ref[...] = jnp.concatenate(idxs, axis=1).T + b * S
    w_ref[...] = jnp.concatenate([x / norm for x in r], axis=1).T


def _three_nn(xyz1, xyz2, BN=2048):
    B, N, _ = xyz1.shape
    S = xyz2.shape[1]
    x2t = jnp.transpose(xyz2, (0, 2, 1))
    NB = N // BN
    idxf, wf = pl.pallas_call(
        functools.partial(_knn_body, S=S),
        grid=(B, NB),
        in_specs=[
            pl.BlockSpec((None, BN, 3), lambda b, i: (b, i, 0)),
            pl.BlockSpec((None, 3, S), lambda b, i: (b, 0, 0)),
        ],
        out_specs=[
            pl.BlockSpec((3, BN), lambda b, i: (0, b * NB + i)),
            pl.BlockSpec((3, BN), lambda b, i: (0, b * NB + i)),
        ],
        out_shape=[
            jax.ShapeDtypeStruct((3, B * N), jnp.int32),
            jax.ShapeDtypeStruct((3, B * N), jnp.float32),
        ],
    )(xyz1, x2t)
    return idxf, wf


def _lane_broadcast(vec, lane_idx):
    return lax.gather(
        vec,
        lane_idx[:, None],
        dimension_numbers=lax.GatherDimensionNumbers(
            offset_dims=(), collapsed_slice_dims=(0,), start_index_map=(0,)
        ),
        slice_sizes=(1,),
        mode=lax.GatherScatterMode.PROMISE_IN_BOUNDS,
    )


def _sc_interpolate(table, idxf, wf):
    BNtot = idxf.shape[1]
    C = table.shape[1]
    NC, NS = 2, 16
    NW = NC * NS
    PW = BNtot // NW
    P = 64
    NCH = PW // P
    CV = C // 16

    mesh = plsc.VectorSubcoreMesh(
        core_axis_name="c", subcore_axis_name="s", num_cores=NC, num_subcores=NS
    )

    @functools.partial(
        pl.kernel,
        mesh=mesh,
        out_type=jax.ShapeDtypeStruct((BNtot, C), jnp.float32),
        scratch_types=[
            pltpu.VMEM((3, PW), jnp.int32),
            pltpu.VMEM((3, PW), jnp.float32),
            pltpu.VMEM((3, P, C), jnp.float32),
            pltpu.VMEM((P, C), jnp.float32),
            pltpu.SemaphoreType.DMA,
        ],
    )
    def interp(table_hbm, idx_hbm, w_hbm, out_hbm, idx_v, w_v, rows_v, out_v, sem):
        wid = lax.axis_index("s") * NC + lax.axis_index("c")
        base = wid * PW
        pltpu.sync_copy(idx_hbm.at[:, pl.ds(base, PW)], idx_v)
        pltpu.sync_copy(w_hbm.at[:, pl.ds(base, PW)], w_v)

        @pl.loop(0, NCH)
        def _chunk(i):
            off = i * P
            cps = [
                pltpu.async_copy(
                    table_hbm.at[idx_v.at[k, pl.ds(off, P)]], rows_v.at[k], sem
                )
                for k in range(3)
            ]
            for cp in cps:
                cp.wait()

            @pl.loop(0, P // 16)
            def _group(g):
                wrow = [w_v[k, pl.ds(off + g * 16, 16)] for k in range(3)]
                for t in range(16):
                    lane = jnp.full((16,), t, jnp.int32)
                    wv = [_lane_broadcast(wrow[k], lane) for k in range(3)]
                    p = g * 16 + t
                    for j in range(CV):
                        sl = pl.ds(j * 16, 16)
                        acc = wv[0] * rows_v[0, p, sl]
                        acc = acc + wv[1] * rows_v[1, p, sl]
                        acc = acc + wv[2] * rows_v[2, p, sl]
                        out_v[p, sl] = acc

            pltpu.sync_copy(out_v, out_hbm.at[pl.ds(base + off, P)])

    return interp(table, idxf, wf)


def _mlp1_body(p1_ref, it_ref, w_ref, z_ref, s_ref):
    x = jnp.concatenate([p1_ref[...], it_ref[...]], axis=1)
    z = jnp.dot(x, w_ref[...], preferred_element_type=jnp.float32)
    z_ref[...] = z.astype(z_ref.dtype)

    @pl.when(pl.program_id(0) == 0)
    def _():
        s_ref[...] = jnp.zeros_like(s_ref)

    s_ref[...] += jnp.concatenate(
        [jnp.sum(z, 0, keepdims=True), jnp.sum(z * z, 0, keepdims=True)], axis=0
    )


def _scale_shift(s_ref, g_ref, b_ref, count):
    mean = s_ref[0:1, :] * (1.0 / count)
    ex2 = s_ref[1:2, :] * (1.0 / count)
    var = ex2 - mean * mean
    scale = g_ref[...] * lax.rsqrt(var + 1e-5)
    shift = b_ref[...] - mean * scale
    return scale, shift


def _mlp_mid_body(s_in_ref, g_ref, b_ref, z_in_ref, w_ref, z_ref, s_ref, *, count):
    scale, shift = _scale_shift(s_in_ref, g_ref, b_ref, count)
    a = jnp.maximum(z_in_ref[...].astype(jnp.float32) * scale + shift, 0.0)
    z = jnp.dot(a, w_ref[...], preferred_element_type=jnp.float32)
    z_ref[...] = z.astype(z_ref.dtype)

    @pl.when(pl.program_id(0) == 0)
    def _():
        s_ref[...] = jnp.zeros_like(s_ref)

    s_ref[...] += jnp.concatenate(
        [jnp.sum(z, 0, keepdims=True), jnp.sum(z * z, 0, keepdims=True)], axis=0
    )


def _final_body(s_in_ref, g_ref, b_ref, z_in_ref, o_ref, *, count):
    scale, shift = _scale_shift(s_in_ref, g_ref, b_ref, count)
    o_ref[...] = jnp.maximum(z_in_ref[...].astype(jnp.float32) * scale + shift, 0.0)


def _mlp1(p1, interp, W1t, BM=2048):
    BNtot, Ca = p1.shape
    Cb = interp.shape[1]
    Cout = W1t.shape[1]
    NB = BNtot // BM
    return pl.pallas_call(
        _mlp1_body,
        grid=(NB,),
        in_specs=[
            pl.BlockSpec((BM, Ca), lambda i: (i, 0)),
            pl.BlockSpec((BM, Cb), lambda i: (i, 0)),
            pl.BlockSpec((Ca + Cb, Cout), lambda i: (0, 0)),
        ],
        out_specs=[
            pl.BlockSpec((BM, Cout), lambda i: (i, 0)),
            pl.BlockSpec((2, Cout), lambda i: (0, 0)),
        ],
        out_shape=[
            jax.ShapeDtypeStruct((BNtot, Cout), jnp.bfloat16),
            jax.ShapeDtypeStruct((2, Cout), jnp.float32),
        ],
    )(p1, interp, W1t)


def _mlp_mid(s_in, g, b, z_in, Wt, BM=2048):
    BNtot, Cin = z_in.shape
    Cout = Wt.shape[1]
    NB = BNtot // BM
    return pl.pallas_call(
        functools.partial(_mlp_mid_body, count=BNtot),
        grid=(NB,),
        in_specs=[
            pl.BlockSpec((2, Cin), lambda i: (0, 0)),
            pl.BlockSpec((1, Cin), lambda i: (0, 0)),
            pl.BlockSpec((1, Cin), lambda i: (0, 0)),
            pl.BlockSpec((BM, Cin), lambda i: (i, 0)),
            pl.BlockSpec((Cin, Cout), lambda i: (0, 0)),
        ],
        out_specs=[
            pl.BlockSpec((BM, Cout), lambda i: (i, 0)),
            pl.BlockSpec((2, Cout), lambda i: (0, 0)),
        ],
        out_shape=[
            jax.ShapeDtypeStruct((BNtot, Cout), jnp.bfloat16),
            jax.ShapeDtypeStruct((2, Cout), jnp.float32),
        ],
    )(s_in, g, b, z_in, Wt)


def _mlp_final(s_in, g, b, z_in, BM=2048):
    BNtot, Cin = z_in.shape
    NB = BNtot // BM
    return pl.pallas_call(
        functools.partial(_final_body, count=BNtot),
        grid=(NB,),
        in_specs=[
            pl.BlockSpec((2, Cin), lambda i: (0, 0)),
            pl.BlockSpec((1, Cin), lambda i: (0, 0)),
            pl.BlockSpec((1, Cin), lambda i: (0, 0)),
            pl.BlockSpec((BM, Cin), lambda i: (i, 0)),
        ],
        out_specs=pl.BlockSpec((BM, Cin), lambda i: (i, 0)),
        out_shape=jax.ShapeDtypeStruct((BNtot, Cin), jnp.float32),
    )(s_in, g, b, z_in)


def kernel(xyz1, xyz2, points1, points2, W1, g1, b1, W2, g2, b2, W3, g3, b3):
    B, N, _ = xyz1.shape
    S = xyz2.shape[1]
    C1 = points1.shape[2]
    C2 = points2.shape[2]

    idxf, wf = _three_nn(xyz1, xyz2)

    table = points2.reshape(B * S, C2)
    interp = _sc_interpolate(table, idxf, wf)

    p1 = points1.reshape(B * N, C1)
    z1, s1 = _mlp1(p1, interp, jnp.transpose(W1))
    z2, s2 = _mlp_mid(s1, g1.reshape(1, -1), b1.reshape(1, -1), z1, jnp.transpose(W2))
    z3, s3 = _mlp_mid(s2, g2.reshape(1, -1), b2.reshape(1, -1), z2, jnp.transpose(W3))
    out = _mlp_final(s3, g3.reshape(1, -1), b3.reshape(1, -1), z3)
    return out.reshape(B, N, -1)
